# baseline (speedup 1.0000x reference)
import functools

import jax
import jax.numpy as jnp
from jax import lax
from jax.experimental import pallas as pl
from jax.experimental.pallas import tpu as pltpu
from jax.experimental.pallas import tpu_sc as plsc

D = 128
NC = 2
NS = 16
NW = NC * NS
CH = 128


def _gather_body(nch, table_hbm, idx_hbm, out_hbm, idx_v, rows_v, sem):
    wid = lax.axis_index("s") * NC + lax.axis_index("c")
    b_per_w = nch * CH
    base = wid * b_per_w
    pltpu.sync_copy(idx_hbm.at[wid], idx_v)
    gathers = [
        pltpu.async_copy(
            table_hbm.at[idx_v.at[j]], rows_v.at[pl.ds(j * CH, CH)], sem
        )
        for j in range(nch)
    ]
    for cp in gathers:
        cp.wait()
    pltpu.sync_copy(rows_v, out_hbm.at[pl.ds(base, b_per_w)])


@functools.lru_cache(maxsize=None)
def _make_sc_gather(batch):
    assert batch % (NW * CH) == 0
    b_per_w = batch // NW
    nch = b_per_w // CH
    mesh = plsc.VectorSubcoreMesh(
        core_axis_name="c", subcore_axis_name="s",
        num_cores=NC, num_subcores=NS,
    )
    return pl.kernel(
        functools.partial(_gather_body, nch),
        out_type=jax.ShapeDtypeStruct((batch, D), jnp.float32),
        mesh=mesh,
        scratch_types=[
            pltpu.VMEM((nch, CH), jnp.int32),
            pltpu.VMEM((b_per_w, D), jnp.float32),
            pltpu.SemaphoreType.DMA,
        ],
    )


def _mlp_compute(x, w1, b1, w2, b2):
    bf = jnp.bfloat16
    h = jnp.dot(x.astype(bf), w1.astype(bf),
                preferred_element_type=jnp.float32)
    h = h + b1
    h = h * jax.nn.sigmoid(h)
    o = jnp.dot(h.astype(bf), w2.astype(bf),
                preferred_element_type=jnp.float32)
    return o + b2


def _mlp_body(x_ref, w1_ref, b1_ref, w2_ref, b2_ref, o_ref):
    o_ref[...] = _mlp_compute(
        x_ref[...], w1_ref[...], b1_ref[...], w2_ref[...], b2_ref[...]
    )


@functools.lru_cache(maxsize=None)
def _make_mlp(batch, bm):
    grid = (batch // bm,)
    return pl.pallas_call(
        _mlp_body,
        grid=grid,
        in_specs=[
            pl.BlockSpec((bm, D), lambda i: (i, 0)),
            pl.BlockSpec((D, D), lambda i: (0, 0)),
            pl.BlockSpec((1, D), lambda i: (0, 0)),
            pl.BlockSpec((D, D), lambda i: (0, 0)),
            pl.BlockSpec((1, D), lambda i: (0, 0)),
        ],
        out_specs=pl.BlockSpec((bm, D), lambda i: (i, 0)),
        out_shape=jax.ShapeDtypeStruct((batch, D), jnp.float32),
        input_output_aliases={0: 0},
    )


@jax.jit
def kernel(context, table, W1, b1, W2, b2):
    batch = context.shape[0]
    idx = context.astype(jnp.int32).reshape(NW, batch // (NW * CH), CH)
    gathered = _make_sc_gather(batch)(table, idx)
    gathered = pltpu.with_memory_space_constraint(
        gathered, pltpu.MemorySpace.HBM
    )
    bm = min(2048, batch)
    return _make_mlp(batch, bm)(
        gathered, W1, b1.reshape(1, D), W2, b2.reshape(1, D)
    )


# speedup vs baseline: 1.2931x; 1.2931x over previous
import functools

import jax
import jax.numpy as jnp
from jax import lax
from jax.experimental import pallas as pl
from jax.experimental.pallas import tpu as pltpu
from jax.experimental.pallas import tpu_sc as plsc

D = 128
NC = 2
NS = 16
NW = NC * NS
CH = 128


def _gather_body(nch, table_hbm, idx_hbm, out_hbm, idx_v, rows_v, sem):
    wid = lax.axis_index("s") * NC + lax.axis_index("c")
    b_per_w = nch * CH
    base = wid * b_per_w
    pltpu.sync_copy(idx_hbm.at[wid], idx_v)
    gathers = [
        pltpu.async_copy(
            table_hbm.at[idx_v.at[j]], rows_v.at[pl.ds(j * CH, CH)], sem
        )
        for j in range(nch)
    ]
    for cp in gathers:
        cp.wait()
    pltpu.sync_copy(rows_v, out_hbm.at[pl.ds(base, b_per_w)])


@functools.lru_cache(maxsize=None)
def _make_sc_gather(batch):
    assert batch % (NW * CH) == 0
    b_per_w = batch // NW
    nch = b_per_w // CH
    mesh = plsc.VectorSubcoreMesh(
        core_axis_name="c", subcore_axis_name="s",
        num_cores=NC, num_subcores=NS,
    )
    return pl.kernel(
        functools.partial(_gather_body, nch),
        out_type=jax.ShapeDtypeStruct((batch, D), jnp.float32),
        mesh=mesh,
        scratch_types=[
            pltpu.VMEM((nch, CH), jnp.int32),
            pltpu.VMEM((b_per_w, D), jnp.float32),
            pltpu.SemaphoreType.DMA,
        ],
    )


def _mlp_compute(x, w1, b1, w2, b2):
    bf = jnp.bfloat16
    h = jnp.dot(x.astype(bf), w1.astype(bf),
                preferred_element_type=jnp.float32)
    h = h + b1
    h = h * jax.nn.sigmoid(h)
    o = jnp.dot(h.astype(bf), w2.astype(bf),
                preferred_element_type=jnp.float32)
    return o + b2


def _mlp_body(x_ref, w1_ref, b1_ref, w2_ref, b2_ref, o_ref):
    o_ref[...] = _mlp_compute(
        x_ref[...], w1_ref[...], b1_ref[...], w2_ref[...], b2_ref[...]
    )


@functools.lru_cache(maxsize=None)
def _make_mlp(batch, bm):
    grid = (batch // bm,)
    return pl.pallas_call(
        _mlp_body,
        grid=grid,
        in_specs=[
            pl.BlockSpec((bm, D), lambda i: (i, 0)),
            pl.BlockSpec((D, D), lambda i: (0, 0)),
            pl.BlockSpec((1, D), lambda i: (0, 0)),
            pl.BlockSpec((D, D), lambda i: (0, 0)),
            pl.BlockSpec((1, D), lambda i: (0, 0)),
        ],
        out_specs=pl.BlockSpec((bm, D), lambda i: (i, 0)),
        out_shape=jax.ShapeDtypeStruct((batch, D), jnp.float32),
        input_output_aliases={0: 0},
    )


@jax.jit
def kernel(context, table, W1, b1, W2, b2):
    batch = context.shape[0]
    idx = context.astype(jnp.int32).reshape(NW, batch // (NW * CH), CH)
    gathered = _make_sc_gather(batch)(table, idx)
    bm = min(8192, batch)
    return _make_mlp(batch, bm)(
        gathered, W1, b1.reshape(1, D), W2, b2.reshape(1, D)
    )
